# baseline (speedup 1.0000x reference)
import jax
import jax.numpy as jnp
from jax import lax
from jax.experimental import pallas as pl
from jax.experimental.pallas import tpu as pltpu
from jax.experimental.pallas import tpu_sc as plsc

_VOCAB = 100000
_SENT = 20
_MEM = 50
_EMB = 32
_BATCH = 1024

_NW = 32
_BATCH_W = _BATCH // _NW
_ROWS_CHUNK = _SENT * _BATCH_W

_SCALE = 4.0 / (_EMB * _SENT)
_W = [float((s + 1) - (_SENT + 1) / 2.0) for s in range(_SENT - 1)]

_VCHUNK = 400
_NVCHUNK = _VOCAB // _VCHUNK
_VCHUNK_W = -(-_NVCHUNK // _NW)


def _transpose_body(embt_hbm, out_hbm, in_v, out_v, isem0, isem1,
                    osem0, osem1):
    wid = lax.axis_index("s") * 2 + lax.axis_index("c")

    isems = (isem0, isem1)
    osems = (osem0, osem1)
    lane = lax.iota(jnp.int32, 16)
    erow = [lane + h * 16 for h in range(2)]

    def voff(k):
        c = jnp.where(wid + k * _NW < _NVCHUNK, wid + k * _NW, wid)
        return pl.multiple_of(c * _VCHUNK, 8)

    def fetch(k, buf):
        return pltpu.make_async_copy(
            embt_hbm.at[:, pl.ds(voff(k), _VCHUNK)],
            in_v.at[buf], isems[buf])

    def store(k, buf):
        return pltpu.make_async_copy(
            out_v.at[buf], out_hbm.at[pl.ds(voff(k), _VCHUNK)],
            osems[buf])

    e_splat = [jnp.full((16,), e, jnp.int32) for e in range(_EMB)]

    def transpose(buf):
        def blk_body(vb, carry):
            v_vec = lane + vb * 16
            for e in range(_EMB):
                plsc.store_scatter(out_v.at[buf], [v_vec, e_splat[e]],
                                   in_v[buf, e, pl.ds(vb * 16, 16)])
            return carry
        lax.fori_loop(0, _VCHUNK // 16, blk_body, 0)

    fetch(0, 0).start()
    fetch(1, 1).start()
    for k in range(_VCHUNK_W):
        buf = k % 2
        fetch(k, buf).wait()
        if k >= 2:
            store(k - 2, buf).wait()
        transpose(buf)
        store(k, buf).start()
        if k + 2 < _VCHUNK_W:
            fetch(k + 2, buf).start()
    for k in range(_VCHUNK_W - 2, _VCHUNK_W):
        store(k, k % 2).wait()


_transpose_call = pl.kernel(
    _transpose_body,
    out_type=jax.ShapeDtypeStruct((_VOCAB, _EMB), jnp.float32),
    mesh=plsc.VectorSubcoreMesh(core_axis_name="c", subcore_axis_name="s"),
    scratch_types=[
        pltpu.VMEM((2, _EMB, _VCHUNK), jnp.float32),
        pltpu.VMEM((2, _VCHUNK, _EMB), jnp.float32),
        pltpu.SemaphoreType.DMA,
        pltpu.SemaphoreType.DMA,
        pltpu.SemaphoreType.DMA,
        pltpu.SemaphoreType.DMA,
    ],
    compiler_params=pltpu.CompilerParams(use_tc_tiling_on_sc=False,
                                         needs_layout_passes=False),
)


def _sc_body(x_hbm, emb_hbm, temp_hbm, out_hbm,
             idx_v, rows_v, out_v, temp_v, gsem0, gsem1):
    wid = lax.axis_index("s") * 2 + lax.axis_index("c")
    b0 = wid * _BATCH_W

    pltpu.sync_copy(x_hbm.at[:, :, pl.ds(b0, _BATCH_W)], idx_v)
    pltpu.sync_copy(temp_hbm, temp_v)

    gsems = (gsem0, gsem1)

    lane = lax.iota(jnp.int32, 16)
    lane_f = lane.astype(jnp.float32)
    cvec = [(lane_f + float(h * 16) - (_EMB - 1) / 2.0) * _SCALE
            for h in range(2)]
    erow = [lane + h * 16 for h in range(2)]

    def fire(m, buf):
        for s in range(_SENT):
            pltpu.make_async_copy(
                emb_hbm.at[idx_v.at[s, m]],
                rows_v.at[buf, pl.ds(s * _BATCH_W, _BATCH_W)],
                gsems[buf],
            ).start()

    def drain(buf):
        pltpu.make_async_copy(
            emb_hbm.at[pl.ds(0, _ROWS_CHUNK)], rows_v.at[buf], gsems[buf]
        ).wait()

    def compute(m, buf):
        def b_body(b, carry):
            b_vec = jnp.full((16,), b, jnp.int32)
            for h in range(2):
                sl = pl.ds(h * 16, 16)
                acc = rows_v[buf, b, sl] * _W[0]
                for s in range(1, _SENT - 1):
                    acc = acc + rows_v[buf, s * _BATCH_W + b, sl] * _W[s]
                res = (acc * cvec[h]
                       + rows_v[buf, (_SENT - 1) * _BATCH_W + b, sl]
                       + temp_v[m, sl])
                plsc.store_scatter(out_v.at[m], [erow[h], b_vec], res)
            return carry
        lax.fori_loop(0, _BATCH_W, b_body, 0)

    fire(0, 0)
    fire(1, 1)

    def loop_body(i, carry):
        for buf in range(2):
            m = 2 * i + buf
            drain(buf)
            compute(m, buf)
            fire(m + 2, buf)
        return carry

    lax.fori_loop(0, _MEM // 2 - 1, loop_body, 0)

    for buf in range(2):
        drain(buf)
        compute(_MEM - 2 + buf, buf)

    pltpu.sync_copy(out_v, out_hbm.at[:, :, pl.ds(b0, _BATCH_W)])


_sc_call = pl.kernel(
    _sc_body,
    out_type=jax.ShapeDtypeStruct((_MEM, _EMB, _BATCH), jnp.float32),
    mesh=plsc.VectorSubcoreMesh(core_axis_name="c", subcore_axis_name="s"),
    scratch_types=[
        pltpu.VMEM((_SENT, _MEM, _BATCH_W), jnp.int32),
        pltpu.VMEM((2, _ROWS_CHUNK, _EMB), jnp.float32),
        pltpu.VMEM((_MEM, _EMB, _BATCH_W), jnp.float32),
        pltpu.VMEM((_MEM, _EMB), jnp.float32),
        pltpu.SemaphoreType.DMA,
        pltpu.SemaphoreType.DMA,
    ],
    compiler_params=pltpu.CompilerParams(use_tc_tiling_on_sc=False,
                                         needs_layout_passes=False),
)


@jax.jit
def kernel(x, emb_table, temporal_table):
    xt = x.astype(jnp.int32).transpose(2, 1, 0)
    table = _transpose_call(emb_table.T)
    out_t = _sc_call(xt, table, temporal_table)
    return out_t.transpose(2, 0, 1)


# speedup vs baseline: 1.2024x; 1.2024x over previous
import jax
import jax.numpy as jnp
from jax import lax
from jax.experimental import pallas as pl
from jax.experimental.pallas import tpu as pltpu
from jax.experimental.pallas import tpu_sc as plsc

_VOCAB = 100000
_SENT = 20
_MEM = 50
_EMB = 32
_BATCH = 1024

_NW = 32
_BATCH_W = _BATCH // _NW
_ROWS_CHUNK = _SENT * _BATCH_W

_SCALE = 4.0 / (_EMB * _SENT)
_W = [float((s + 1) - (_SENT + 1) / 2.0) for s in range(_SENT - 1)]

_VCHUNK = 400
_NVCHUNK = _VOCAB // _VCHUNK
_VCHUNK_W = -(-_NVCHUNK // _NW)


def _transpose_body(embt_hbm, out_hbm, in_v, out_v, isem0, isem1,
                    osem0, osem1):
    wid = lax.axis_index("s") * 2 + lax.axis_index("c")

    isems = (isem0, isem1)
    osems = (osem0, osem1)
    lane = lax.iota(jnp.int32, 16)
    erow = [lane + h * 16 for h in range(2)]

    def voff(k):
        c = jnp.where(wid + k * _NW < _NVCHUNK, wid + k * _NW, wid)
        return pl.multiple_of(c * _VCHUNK, 8)

    def fetch(k, buf):
        return pltpu.make_async_copy(
            embt_hbm.at[:, pl.ds(voff(k), _VCHUNK)],
            in_v.at[buf], isems[buf])

    def store(k, buf):
        return pltpu.make_async_copy(
            out_v.at[buf], out_hbm.at[pl.ds(voff(k), _VCHUNK)],
            osems[buf])

    e_splat = [jnp.full((16,), e, jnp.int32) for e in range(_EMB)]

    def transpose(buf):
        def blk_body(vb, carry):
            v_vec = lane + vb * 16
            for e in range(_EMB):
                plsc.store_scatter(out_v.at[buf], [v_vec, e_splat[e]],
                                   in_v[buf, e, pl.ds(vb * 16, 16)])
            return carry
        lax.fori_loop(0, _VCHUNK // 16, blk_body, 0)

    fetch(0, 0).start()
    fetch(1, 1).start()
    for k in range(_VCHUNK_W):
        buf = k % 2
        fetch(k, buf).wait()
        if k >= 2:
            store(k - 2, buf).wait()
        transpose(buf)
        store(k, buf).start()
        if k + 2 < _VCHUNK_W:
            fetch(k + 2, buf).start()
    for k in range(_VCHUNK_W - 2, _VCHUNK_W):
        store(k, k % 2).wait()


_transpose_call = pl.kernel(
    _transpose_body,
    out_type=jax.ShapeDtypeStruct((_VOCAB, _EMB), jnp.float32),
    mesh=plsc.VectorSubcoreMesh(core_axis_name="c", subcore_axis_name="s"),
    scratch_types=[
        pltpu.VMEM((2, _EMB, _VCHUNK), jnp.float32),
        pltpu.VMEM((2, _VCHUNK, _EMB), jnp.float32),
        pltpu.SemaphoreType.DMA,
        pltpu.SemaphoreType.DMA,
        pltpu.SemaphoreType.DMA,
        pltpu.SemaphoreType.DMA,
    ],
    compiler_params=pltpu.CompilerParams(use_tc_tiling_on_sc=False,
                                         needs_layout_passes=False),
)


def _sc_body(x_hbm, emb_hbm, temp_hbm, out_hbm,
             idx_v, rows_v, out_v, temp_v, gsem0, gsem1):
    wid = lax.axis_index("s") * 2 + lax.axis_index("c")
    b0 = wid * _BATCH_W

    pltpu.sync_copy(x_hbm.at[:, :, pl.ds(b0, _BATCH_W)], idx_v)
    pltpu.sync_copy(temp_hbm, temp_v)

    gsems = (gsem0, gsem1)

    lane = lax.iota(jnp.int32, 16)
    lane_f = lane.astype(jnp.float32)
    cvec = [(lane_f + float(h * 16) - (_EMB - 1) / 2.0) * _SCALE
            for h in range(2)]
    erow = [lane + h * 16 for h in range(2)]

    def fire(m, buf):
        for s in range(_SENT):
            pltpu.make_async_copy(
                emb_hbm.at[idx_v.at[s, m]],
                rows_v.at[buf, pl.ds(s * _BATCH_W, _BATCH_W)],
                gsems[buf],
            ).start()

    def drain(buf):
        pltpu.make_async_copy(
            emb_hbm.at[pl.ds(0, _ROWS_CHUNK)], rows_v.at[buf], gsems[buf]
        ).wait()

    def compute(m, buf):
        def b_body(b, carry):
            for h in range(2):
                sl = pl.ds(h * 16, 16)
                acc = rows_v[buf, b, sl] * _W[0]
                for s in range(1, _SENT - 1):
                    acc = acc + rows_v[buf, s * _BATCH_W + b, sl] * _W[s]
                out_v[m, b, sl] = (acc * cvec[h]
                                   + rows_v[buf, (_SENT - 1) * _BATCH_W + b,
                                            sl]
                                   + temp_v[m, sl])
            return carry
        lax.fori_loop(0, _BATCH_W, b_body, 0)

    fire(0, 0)
    fire(1, 1)

    def loop_body(i, carry):
        for buf in range(2):
            m = 2 * i + buf
            drain(buf)
            compute(m, buf)
            fire(m + 2, buf)
        return carry

    lax.fori_loop(0, _MEM // 2 - 1, loop_body, 0)

    for buf in range(2):
        drain(buf)
        compute(_MEM - 2 + buf, buf)

    pltpu.sync_copy(out_v, out_hbm.at[:, pl.ds(b0, _BATCH_W), :])


_sc_call = pl.kernel(
    _sc_body,
    out_type=jax.ShapeDtypeStruct((_MEM, _BATCH, _EMB), jnp.float32),
    mesh=plsc.VectorSubcoreMesh(core_axis_name="c", subcore_axis_name="s"),
    scratch_types=[
        pltpu.VMEM((_SENT, _MEM, _BATCH_W), jnp.int32),
        pltpu.VMEM((2, _ROWS_CHUNK, _EMB), jnp.float32),
        pltpu.VMEM((_MEM, _BATCH_W, _EMB), jnp.float32),
        pltpu.VMEM((_MEM, _EMB), jnp.float32),
        pltpu.SemaphoreType.DMA,
        pltpu.SemaphoreType.DMA,
    ],
    compiler_params=pltpu.CompilerParams(use_tc_tiling_on_sc=False,
                                         needs_layout_passes=False),
)


@jax.jit
def kernel(x, emb_table, temporal_table):
    xt = x.astype(jnp.int32).transpose(2, 1, 0)
    out_t = _sc_call(xt, emb_table, temporal_table)
    return out_t.transpose(1, 0, 2)


# speedup vs baseline: 1.3156x; 1.0941x over previous
import jax
import jax.numpy as jnp
from jax import lax
from jax.experimental import pallas as pl
from jax.experimental.pallas import tpu as pltpu
from jax.experimental.pallas import tpu_sc as plsc

_VOCAB = 100000
_SENT = 20
_MEM = 50
_EMB = 32
_BATCH = 1024

_NW = 32
_BATCH_W = _BATCH // _NW
_ROWS_CHUNK = _SENT * _BATCH_W

_SCALE = 4.0 / (_EMB * _SENT)
_W = [float((s + 1) - (_SENT + 1) / 2.0) for s in range(_SENT - 1)]

_VCHUNK = 400
_NVCHUNK = _VOCAB // _VCHUNK
_VCHUNK_W = -(-_NVCHUNK // _NW)


def _transpose_body(embt_hbm, out_hbm, in_v, out_v, isem0, isem1,
                    osem0, osem1):
    wid = lax.axis_index("s") * 2 + lax.axis_index("c")

    isems = (isem0, isem1)
    osems = (osem0, osem1)
    lane = lax.iota(jnp.int32, 16)
    erow = [lane + h * 16 for h in range(2)]

    def voff(k):
        c = jnp.where(wid + k * _NW < _NVCHUNK, wid + k * _NW, wid)
        return pl.multiple_of(c * _VCHUNK, 8)

    def fetch(k, buf):
        return pltpu.make_async_copy(
            embt_hbm.at[:, pl.ds(voff(k), _VCHUNK)],
            in_v.at[buf], isems[buf])

    def store(k, buf):
        return pltpu.make_async_copy(
            out_v.at[buf], out_hbm.at[pl.ds(voff(k), _VCHUNK)],
            osems[buf])

    e_splat = [jnp.full((16,), e, jnp.int32) for e in range(_EMB)]

    def transpose(buf):
        def blk_body(vb, carry):
            v_vec = lane + vb * 16
            for e in range(_EMB):
                plsc.store_scatter(out_v.at[buf], [v_vec, e_splat[e]],
                                   in_v[buf, e, pl.ds(vb * 16, 16)])
            return carry
        lax.fori_loop(0, _VCHUNK // 16, blk_body, 0)

    fetch(0, 0).start()
    fetch(1, 1).start()
    for k in range(_VCHUNK_W):
        buf = k % 2
        fetch(k, buf).wait()
        if k >= 2:
            store(k - 2, buf).wait()
        transpose(buf)
        store(k, buf).start()
        if k + 2 < _VCHUNK_W:
            fetch(k + 2, buf).start()
    for k in range(_VCHUNK_W - 2, _VCHUNK_W):
        store(k, k % 2).wait()


_transpose_call = pl.kernel(
    _transpose_body,
    out_type=jax.ShapeDtypeStruct((_VOCAB, _EMB), jnp.float32),
    mesh=plsc.VectorSubcoreMesh(core_axis_name="c", subcore_axis_name="s"),
    scratch_types=[
        pltpu.VMEM((2, _EMB, _VCHUNK), jnp.float32),
        pltpu.VMEM((2, _VCHUNK, _EMB), jnp.float32),
        pltpu.SemaphoreType.DMA,
        pltpu.SemaphoreType.DMA,
        pltpu.SemaphoreType.DMA,
        pltpu.SemaphoreType.DMA,
    ],
    compiler_params=pltpu.CompilerParams(use_tc_tiling_on_sc=False,
                                         needs_layout_passes=False),
)


def _sc_body(x_hbm, emb_hbm, temp_hbm, out_hbm,
             idx_v, rows_v, out_v, pack_v, temp_v,
             gsem0, gsem1, osem0, osem1):
    wid = lax.axis_index("s") * 2 + lax.axis_index("c")
    b0 = wid * _BATCH_W

    pltpu.sync_copy(x_hbm.at[:, :, pl.ds(b0, _BATCH_W)], idx_v)
    pltpu.sync_copy(temp_hbm, temp_v)

    gsems = (gsem0, gsem1)
    osems = (osem0, osem1)

    lane = lax.iota(jnp.int32, 16)
    lane_f = lane.astype(jnp.float32)
    cvec = [(lane_f + float(h * 16) - (_EMB - 1) / 2.0) * _SCALE
            for h in range(2)]
    erow = [lane + h * 16 for h in range(2)]

    def fire(m, buf):
        for s in range(_SENT):
            pltpu.make_async_copy(
                emb_hbm.at[idx_v.at[s, m]],
                rows_v.at[buf, pl.ds(s * _BATCH_W, _BATCH_W)],
                gsems[buf],
            ).start()

    def drain(buf):
        pltpu.make_async_copy(
            emb_hbm.at[pl.ds(0, _ROWS_CHUNK)], rows_v.at[buf], gsems[buf]
        ).wait()

    def compute(m, buf):
        def b_body(b, carry):
            b_vec = jnp.full((16,), b, jnp.int32)
            for h in range(2):
                sl = pl.ds(h * 16, 16)
                acc = rows_v[buf, b, sl] * _W[0]
                for s in range(1, _SENT - 1):
                    acc = acc + rows_v[buf, s * _BATCH_W + b, sl] * _W[s]
                res = (acc * cvec[h]
                       + rows_v[buf, (_SENT - 1) * _BATCH_W + b, sl]
                       + temp_v[m, sl])
                plsc.store_scatter(out_v.at[buf], [erow[h], b_vec], res)
            return carry
        lax.fori_loop(0, _BATCH_W, b_body, 0)
        for e in range(_EMB):
            for h in range(2):
                pack_v[buf, e, pl.ds(h * 16, 16)] = (
                    out_v[buf, e, pl.ds(h * 16, 16)])

    def store(m, buf):
        return pltpu.make_async_copy(
            pack_v.at[buf], out_hbm.at[m, :, pl.ds(b0, _BATCH_W)],
            osems[buf])

    fire(0, 0)
    fire(1, 1)
    for buf in range(2):
        drain(buf)
        compute(buf, buf)
        store(buf, buf).start()
        fire(buf + 2, buf)

    def loop_body(i, carry):
        for buf in range(2):
            m = 2 * i + buf
            drain(buf)
            store(m - 2, buf).wait()
            compute(m, buf)
            store(m, buf).start()
            fire(m + 2, buf)
        return carry

    lax.fori_loop(1, _MEM // 2 - 1, loop_body, 0)

    for buf in range(2):
        m = _MEM - 2 + buf
        drain(buf)
        store(m - 2, buf).wait()
        compute(m, buf)
        store(m, buf).start()

    for buf in range(2):
        store(_MEM - 2 + buf, buf).wait()


_sc_call = pl.kernel(
    _sc_body,
    out_type=jax.ShapeDtypeStruct((_MEM, _EMB, _BATCH), jnp.float32),
    mesh=plsc.VectorSubcoreMesh(core_axis_name="c", subcore_axis_name="s"),
    scratch_types=[
        pltpu.VMEM((_SENT, _MEM, _BATCH_W), jnp.int32),
        pltpu.VMEM((2, _ROWS_CHUNK, _EMB), jnp.float32),
        pltpu.VMEM((2, _EMB, _BATCH_W + 1), jnp.float32),
        pltpu.VMEM((2, _EMB, _BATCH_W), jnp.float32),
        pltpu.VMEM((_MEM, _EMB), jnp.float32),
        pltpu.SemaphoreType.DMA,
        pltpu.SemaphoreType.DMA,
        pltpu.SemaphoreType.DMA,
        pltpu.SemaphoreType.DMA,
    ],
    compiler_params=pltpu.CompilerParams(use_tc_tiling_on_sc=False,
                                         needs_layout_passes=False),
)


@jax.jit
def kernel(x, emb_table, temporal_table):
    xt = x.astype(jnp.int32).transpose(2, 1, 0)
    out_t = _sc_call(xt, emb_table, temporal_table)
    return out_t.transpose(2, 0, 1)
